# speedup vs baseline: 8.8323x; 8.8323x over previous
import functools

import jax
import jax.numpy as jnp
from jax import lax
from jax.experimental import pallas as pl
from jax.experimental.pallas import tpu as pltpu

_R = 64


def _body(idx_ref, tab_ref, out_ref):
    idx = idx_ref[...]
    tab = tab_ref[...]
    iota = lax.broadcasted_iota(jnp.int32, (1, 1, 10), 2)
    oh = (idx[:, :, None] == iota).astype(jnp.float32)
    oh2 = oh.reshape(_R * 200, 10)
    out = jnp.dot(oh2, tab, preferred_element_type=jnp.float32)
    out_ref[...] = out.reshape(_R, 200, 12)


def kernel(inputs, table):
    n_rows = inputs.shape[0]
    grid = (n_rows // _R,)
    return pl.pallas_call(
        _body,
        grid=grid,
        in_specs=[
            pl.BlockSpec((_R, 200), lambda i: (i, 0)),
            pl.BlockSpec((10, 12), lambda i: (0, 0)),
        ],
        out_specs=pl.BlockSpec((_R, 200, 12), lambda i: (i, 0, 0)),
        out_shape=jax.ShapeDtypeStruct((n_rows, 200, 12), jnp.float32),
    )(inputs, table)
